# Initial kernel scaffold; baseline (speedup 1.0000x reference)
import jax
import jax.numpy as jnp
from jax.experimental import pallas as pl


def kernel(tokens, types, pos_table, tok_table, type_table):
    raise NotImplementedError("write your pallas kernel here")


# speedup vs baseline: 1.6427x; 1.6427x over previous
import functools

import jax
import jax.numpy as jnp
from jax import lax
from jax.experimental import pallas as pl
from jax.experimental.pallas import tpu as pltpu
from jax.experimental.pallas import tpu_sc as plsc

SEQ = 2048
D = 128
B = 16
NC = 2
NS = 16
NW = NC * NS
LBLK = SEQ // NW
KV = D // 16


def _emb_body(tokens_hbm, types_hbm, pos_hbm, tok_tbl_hbm, typ_tbl_hbm,
              out_hbm, tok_idx, typ_idx, pos_v, typ_v, fused_v, buf_v, sem):
    wid = lax.axis_index("s") * NC + lax.axis_index("c")
    l0 = wid * LBLK
    l0a = (wid // 2) * 128
    coff = (wid % 2) * LBLK

    c1 = pltpu.async_copy(tokens_hbm.at[:, pl.ds(l0a, 128)], tok_idx, sem)
    c2 = pltpu.async_copy(types_hbm.at[:, pl.ds(l0a, 128)], typ_idx, sem)
    c3 = pltpu.async_copy(pos_hbm.at[pl.ds(l0, LBLK)], pos_v, sem)
    c4 = pltpu.async_copy(typ_tbl_hbm, typ_v, sem)
    c1.wait()
    c2.wait()
    c3.wait()
    c4.wait()

    def fuse_row(r, carry):
        for t in range(2):
            for k in range(KV):
                s = pl.ds(k * 16, 16)
                fused_v[t, r, s] = pos_v[r, s] + typ_v[t, s]
        return carry

    lax.fori_loop(0, LBLK, fuse_row, 0)

    def batch_body(b, carry):
        pltpu.async_copy(
            tok_tbl_hbm.at[tok_idx.at[b, pl.ds(coff, LBLK)]], buf_v, sem
        ).wait()

        def add_group(g, inner):
            base = g * 16
            tvec = typ_idx[b, pl.ds(coff + base, 16)]
            for jj in range(16):
                t = tvec[jj]
                r = base + jj
                for k in range(KV):
                    s = pl.ds(k * 16, 16)
                    buf_v[r, s] = buf_v[r, s] + fused_v[t, r, s]
            return inner

        lax.fori_loop(0, LBLK // 16, add_group, 0)
        pltpu.sync_copy(buf_v, out_hbm.at[pl.ds(b * SEQ + l0, LBLK)])
        return carry

    lax.fori_loop(0, B, batch_body, 0)


def kernel(tokens, types, pos_table, tok_table, type_table):
    mesh = plsc.VectorSubcoreMesh(
        core_axis_name="c", subcore_axis_name="s", num_cores=NC, num_subcores=NS
    )
    run = functools.partial(
        pl.kernel,
        mesh=mesh,
        out_type=jax.ShapeDtypeStruct((B * SEQ, D), jnp.float32),
        scratch_types=[
            pltpu.VMEM((B, 128), jnp.int32),
            pltpu.VMEM((B, 128), jnp.int32),
            pltpu.VMEM((LBLK, D), jnp.float32),
            pltpu.VMEM((2, D), jnp.float32),
            pltpu.VMEM((2, LBLK, D), jnp.float32),
            pltpu.VMEM((LBLK, D), jnp.float32),
            pltpu.SemaphoreType.DMA,
        ],
    )(_emb_body)
    out = run(tokens, types, pos_table, tok_table, type_table)
    return out.reshape(B, SEQ, D)


# speedup vs baseline: 1.9753x; 1.2025x over previous
import functools

import jax
import jax.numpy as jnp
from jax import lax
from jax.experimental import pallas as pl
from jax.experimental.pallas import tpu as pltpu
from jax.experimental.pallas import tpu_sc as plsc

SEQ = 2048
D = 128
B = 16
NC = 2
NS = 16
NW = NC * NS
LBLK = SEQ // NW
KV = D // 16


def _emb_body(tokens_hbm, types_hbm, pos_hbm, tok_tbl_hbm, typ_tbl_hbm,
              out_hbm, tok_idx, typ_idx, pos_v, typ_v, fused_v,
              buf0, buf1, obuf0, obuf1, ssem, gsem0, gsem1, osem0, osem1):
    wid = lax.axis_index("s") * NC + lax.axis_index("c")
    l0 = wid * LBLK
    l0a = (wid // 2) * 128
    coff = (wid % 2) * LBLK

    c1 = pltpu.async_copy(tokens_hbm.at[:, pl.ds(l0a, 128)], tok_idx, ssem)
    c2 = pltpu.async_copy(types_hbm.at[:, pl.ds(l0a, 128)], typ_idx, ssem)
    c3 = pltpu.async_copy(pos_hbm.at[pl.ds(l0, LBLK)], pos_v, ssem)
    c4 = pltpu.async_copy(typ_tbl_hbm, typ_v, ssem)
    c1.wait()

    def gather(b, buf, gsem):
        return pltpu.async_copy(
            tok_tbl_hbm.at[tok_idx.at[b, pl.ds(coff, LBLK)]], buf, gsem
        )

    gather(0, buf0, gsem0)
    c2.wait()
    c3.wait()
    c4.wait()

    def fuse_row(r, carry):
        for t in range(2):
            for k in range(KV):
                s = pl.ds(k * 16, 16)
                fused_v[t, r, s] = pos_v[r, s] + typ_v[t, s]
        return carry

    lax.fori_loop(0, LBLK, fuse_row, 0)

    def add_batch(b, buf, obuf):
        def add_group(g, inner):
            base = g * 16
            tvec = typ_idx[b, pl.ds(coff + base, 16)]
            for jj in range(16):
                t = tvec[jj]
                r = base + jj
                for k in range(KV):
                    s = pl.ds(k * 16, 16)
                    obuf[r, s] = buf[r, s] + fused_v[t, r, s]
            return inner

        lax.fori_loop(0, LBLK // 16, add_group, 0)

    def out_copy(b, obuf, osem):
        return pltpu.async_copy(
            obuf, out_hbm.at[pl.ds(b * SEQ + l0, LBLK)], osem
        )

    def half(i, b, buf, obuf, gsem, osem, gsem_other, buf_other):
        pltpu.make_async_copy(
            tok_tbl_hbm.at[tok_idx.at[b, pl.ds(coff, LBLK)]], buf, gsem
        ).wait()

        @pl.when(i > 0)
        def _():
            pltpu.make_async_copy(
                obuf, out_hbm.at[pl.ds(b * SEQ + l0, LBLK)], osem
            ).wait()

        add_batch(b, buf, obuf)
        out_copy(b, obuf, osem)

    def pair_body(i, carry):
        b0 = 2 * i
        b1 = b0 + 1
        gather(b1, buf1, gsem1)
        half(i, b0, buf0, obuf0, gsem0, osem0, gsem1, buf1)

        @pl.when(i < B // 2 - 1)
        def _():
            gather(b0 + 2, buf0, gsem0)

        half(i, b1, buf1, obuf1, gsem1, osem1, gsem0, buf0)
        return carry

    lax.fori_loop(0, B // 2, pair_body, 0)

    pltpu.make_async_copy(obuf0, out_hbm.at[pl.ds(l0, LBLK)], osem0).wait()
    pltpu.make_async_copy(obuf1, out_hbm.at[pl.ds(l0, LBLK)], osem1).wait()


def kernel(tokens, types, pos_table, tok_table, type_table):
    mesh = plsc.VectorSubcoreMesh(
        core_axis_name="c", subcore_axis_name="s", num_cores=NC, num_subcores=NS
    )
    run = functools.partial(
        pl.kernel,
        mesh=mesh,
        out_type=jax.ShapeDtypeStruct((B * SEQ, D), jnp.float32),
        scratch_types=[
            pltpu.VMEM((B, 128), jnp.int32),
            pltpu.VMEM((B, 128), jnp.int32),
            pltpu.VMEM((LBLK, D), jnp.float32),
            pltpu.VMEM((2, D), jnp.float32),
            pltpu.VMEM((2, LBLK, D), jnp.float32),
            pltpu.VMEM((LBLK, D), jnp.float32),
            pltpu.VMEM((LBLK, D), jnp.float32),
            pltpu.VMEM((LBLK, D), jnp.float32),
            pltpu.VMEM((LBLK, D), jnp.float32),
            pltpu.SemaphoreType.DMA,
            pltpu.SemaphoreType.DMA,
            pltpu.SemaphoreType.DMA,
            pltpu.SemaphoreType.DMA,
            pltpu.SemaphoreType.DMA,
        ],
    )(_emb_body)
    out = run(tokens, types, pos_table, tok_table, type_table)
    return out.reshape(B, SEQ, D)
